# baseline (speedup 1.0000x reference)
import jax
import jax.numpy as jnp
from jax.experimental import pallas as pl
from jax.experimental.pallas import tpu as pltpu

_C = 16


def _leaky(x, slope):
    return jnp.maximum(x, slope * x)


def _dot_t(a, b):
    return jax.lax.dot_general(a, b, (((1,), (1,)), ((), ())),
                               preferred_element_type=jnp.float32)


def _dot_c(a, b):
    return jax.lax.dot_general(a, b, (((0,), (0,)), ((), ())),
                               preferred_element_type=jnp.float32)


def _graph_stage(rep, g2, w1_ref, al1_ref, ar1_ref, b1_ref,
                 w2_ref, al2_ref, ar2_ref, b2_ref, wc_ref, bc_ref):
    rr = _dot_t(rep, rep)
    jio = jax.lax.broadcasted_iota(jnp.int32, (_C, _C), 1)
    iio = jax.lax.broadcasted_iota(jnp.int32, (_C, _C), 0)
    eye = (jio == iio).astype(jnp.float32)
    n2_row = jnp.sum(rr * eye, axis=0, keepdims=True)
    n2_col = jnp.sum(rr * eye, axis=1, keepdims=True)
    nrm_row = jnp.maximum(jnp.sqrt(n2_row), 1e-8)
    nrm_col = jnp.maximum(jnp.sqrt(n2_col), 1e-8)
    simg = rr / (nrm_col * nrm_row) + g2
    rmax = jnp.max(simg, axis=1, keepdims=True)
    dsel = jnp.min(jnp.where(simg == rmax, jio, _C),
                   axis=1, keepdims=True)
    mcnt = (jio == dsel).astype(jnp.float32) + eye
    mask = mcnt > 0.0

    def gat(h, w_ref, al_ref, ar_ref, b_ref):
        z = jnp.dot(h, w_ref[...], preferred_element_type=jnp.float32)
        el = jnp.sum(z * al_ref[...], axis=1, keepdims=True)
        er = _dot_t(ar_ref[...], z)
        e = _leaky(el + er, 0.2)
        m = jnp.max(jnp.where(mask, e, -jnp.inf), axis=0, keepdims=True)
        ex = jnp.where(mask, mcnt * jnp.exp(e - m), 0.0)
        a = ex / jnp.sum(ex, axis=0, keepdims=True)
        return _dot_c(a, z) + b_ref[...]

    h = _leaky(gat(rep, w1_ref, al1_ref, ar1_ref, b1_ref), 0.01)
    h = _leaky(gat(h, w2_ref, al2_ref, ar2_ref, b2_ref), 0.01)
    pooled = jnp.mean(h, axis=0, keepdims=True)
    o = jnp.dot(pooled, wc_ref[...], preferred_element_type=jnp.float32)
    return _leaky(o + bc_ref[...], 0.01)


def _body(inst_ref, wimg_ref, bimg_ref, cent_ref, wattn_ref, battn_ref,
          g_ref, ones_ref, g2_ref, w1_ref, al1_ref, ar1_ref, b1_ref,
          w2_ref, al2_ref, ar2_ref, b2_ref, wc_ref, bc_ref,
          out_ref, rep_ref, best_ref):
    step = pl.program_id(0)
    nsteps = pl.num_programs(0)

    @pl.when(step == 0)
    def _init():
        best_ref[...] = jnp.full(best_ref.shape, -jnp.inf, jnp.float32)

    cent = cent_ref[...]
    cn = jnp.sum(cent * cent, axis=1, keepdims=True)
    tile = inst_ref.shape[0]
    nchunks = 1
    chunk = tile // nchunks

    w = wimg_ref[...]
    w_hi = w.astype(jnp.bfloat16)
    w_lo = (w - w_hi.astype(jnp.float32)).astype(jnp.bfloat16)

    def _chunk(lo):
        a = inst_ref[pl.ds(lo, chunk), :]
        a_hi = a.astype(jnp.bfloat16)
        a_lo = (a - a_hi.astype(jnp.float32)).astype(jnp.bfloat16)
        xi = jnp.dot(a_hi, w_hi, preferred_element_type=jnp.float32)
        xi = xi + (jnp.dot(a_hi, w_lo, preferred_element_type=jnp.float32)
                   + jnp.dot(a_lo, w_hi, preferred_element_type=jnp.float32))
        xi = _leaky(xi + bimg_ref[...], 0.01)
        t = _dot_t(cent, xi)
        xn = _dot_t(ones_ref[...], xi * xi)
        ns = xn - 2.0 * t + cn
        num = 1.0 / (1.0 + ns)
        ca = num / jnp.sum(num, axis=0, keepdims=True)
        s = _dot_t(wattn_ref[...], xi)
        logits = ca * s + battn_ref[...] + g_ref[0][:, lo:lo + chunk]
        col = jax.lax.broadcasted_iota(jnp.int32, logits.shape, 1)
        lmax = jnp.max(logits, axis=1, keepdims=True)
        lidx = jnp.min(jnp.where(logits == lmax, col, chunk),
                       axis=1, keepdims=True)
        sel = (col == lidx).astype(jnp.float32)
        return lmax, jnp.dot(sel, xi, preferred_element_type=jnp.float32)

    lmax, cand = _chunk(0)
    for k in range(1, nchunks):
        lm2, c2 = _chunk(k * chunk)
        upd = lm2 > lmax
        cand = jnp.where(upd, c2, cand)
        lmax = jnp.maximum(lmax, lm2)

    best = best_ref[...]
    improved = lmax > best
    best_ref[...] = jnp.maximum(best, lmax)
    rep_ref[...] = jnp.where(improved, cand, rep_ref[...])

    @pl.when(step == nsteps - 1)
    def _finish():
        out_ref[...] = _graph_stage(
            rep_ref[...], g2_ref[...], w1_ref, al1_ref, ar1_ref, b1_ref,
            w2_ref, al2_ref, ar2_ref, b2_ref, wc_ref, bc_ref)


def kernel(bags, W_img, b_img, centers, W_attn, b_attn,
           W1, al1, ar1, b1, W2, al2, ar2, b2, Wc, bc):
    inst = bags[0]
    n, d = inst.shape
    h = W1.shape[1]
    ncls = Wc.shape[1]

    tile = next((t for t in range(min(n, 2048), 7, -1)
                 if n % t == 0 and t % 8 == 0), None)
    pad = 0
    if tile is None:
        tile = 2048
        pad = (-n) % tile
    grid = (n + pad) // tile

    def _noise():
        gk = jax.random.key(1)
        g1 = jax.vmap(
            lambda i: jax.random.gumbel(jax.random.fold_in(gk, i), (n,),
                                        jnp.float32))(jnp.arange(_C))
        g1 = jnp.pad(g1, ((0, 0), (0, pad)), constant_values=-jnp.inf)
        g1 = g1.reshape(_C, grid, tile).transpose(1, 0, 2)
        g2 = jax.random.gumbel(jax.random.fold_in(gk, 1000), (_C, _C),
                               jnp.float32)
        return g1, g2

    try:
        with jax.ensure_compile_time_eval():
            g1, g2 = _noise()
    except Exception:
        g1, g2 = _noise()

    if pad:
        inst = jnp.pad(inst, ((0, pad), (0, 0)))

    fixed = lambda i: (0, 0)
    out = pl.pallas_call(
        _body,
        grid=(grid,),
        in_specs=[
            pl.BlockSpec((tile, d), lambda i: (i, 0)),
            pl.BlockSpec((d, d), fixed),
            pl.BlockSpec((1, d), fixed),
            pl.BlockSpec((_C, d), fixed),
            pl.BlockSpec((1, d), fixed),
            pl.BlockSpec((1, 1), fixed),
            pl.BlockSpec((1, _C, tile), lambda i: (i, 0, 0)),
            pl.BlockSpec((1, d), fixed),
            pl.BlockSpec((_C, _C), fixed),
            pl.BlockSpec((d, h), fixed),
            pl.BlockSpec((1, h), fixed),
            pl.BlockSpec((1, h), fixed),
            pl.BlockSpec((1, h), fixed),
            pl.BlockSpec((h, h), fixed),
            pl.BlockSpec((1, h), fixed),
            pl.BlockSpec((1, h), fixed),
            pl.BlockSpec((1, h), fixed),
            pl.BlockSpec((h, ncls), fixed),
            pl.BlockSpec((1, ncls), fixed),
        ],
        out_specs=pl.BlockSpec((1, ncls), fixed),
        out_shape=jax.ShapeDtypeStruct((1, ncls), jnp.float32),
        scratch_shapes=[pltpu.VMEM((_C, d), jnp.float32),
                        pltpu.VMEM((_C, 1), jnp.float32)],
        compiler_params=pltpu.CompilerParams(
            dimension_semantics=("arbitrary",)),
    )(inst, W_img, b_img.reshape(1, d), centers, W_attn.reshape(1, d),
      b_attn.reshape(1, 1), g1, jnp.ones((1, d), jnp.float32),
      g2, W1, al1.reshape(1, h), ar1.reshape(1, h), b1.reshape(1, h),
      W2, al2.reshape(1, h), ar2.reshape(1, h), b2.reshape(1, h),
      Wc, bc.reshape(1, ncls))
    return out[0]


# speedup vs baseline: 1.4602x; 1.4602x over previous
import jax
import jax.numpy as jnp
from jax.experimental import pallas as pl
from jax.experimental.pallas import tpu as pltpu

_C = 16


def _leaky(x, slope):
    return jnp.maximum(x, slope * x)


def _dot_t(a, b):
    return jax.lax.dot_general(a, b, (((1,), (1,)), ((), ())),
                               preferred_element_type=jnp.float32)


def _dot_c(a, b):
    return jax.lax.dot_general(a, b, (((0,), (0,)), ((), ())),
                               preferred_element_type=jnp.float32)


def _graph_stage(rep, g2, w1_ref, al1_ref, ar1_ref, b1_ref,
                 w2_ref, al2_ref, ar2_ref, b2_ref, wc_ref, bc_ref):
    rr = _dot_t(rep, rep)
    jio = jax.lax.broadcasted_iota(jnp.int32, (_C, _C), 1)
    iio = jax.lax.broadcasted_iota(jnp.int32, (_C, _C), 0)
    eye = (jio == iio).astype(jnp.float32)
    n2_row = jnp.sum(rr * eye, axis=0, keepdims=True)
    n2_col = jnp.sum(rr * eye, axis=1, keepdims=True)
    nrm_row = jnp.maximum(jnp.sqrt(n2_row), 1e-8)
    nrm_col = jnp.maximum(jnp.sqrt(n2_col), 1e-8)
    simg = rr / (nrm_col * nrm_row) + g2
    rmax = jnp.max(simg, axis=1, keepdims=True)
    dsel = jnp.min(jnp.where(simg == rmax, jio, _C),
                   axis=1, keepdims=True)
    mcnt = (jio == dsel).astype(jnp.float32) + eye
    mask = mcnt > 0.0

    def gat(h, w_ref, al_ref, ar_ref, b_ref):
        z = jnp.dot(h, w_ref[...], preferred_element_type=jnp.float32)
        el = jnp.sum(z * al_ref[...], axis=1, keepdims=True)
        er = _dot_t(ar_ref[...], z)
        e = _leaky(el + er, 0.2)
        m = jnp.max(jnp.where(mask, e, -jnp.inf), axis=0, keepdims=True)
        ex = jnp.where(mask, mcnt * jnp.exp(e - m), 0.0)
        a = ex / jnp.sum(ex, axis=0, keepdims=True)
        return _dot_c(a, z) + b_ref[...]

    h = _leaky(gat(rep, w1_ref, al1_ref, ar1_ref, b1_ref), 0.01)
    h = _leaky(gat(h, w2_ref, al2_ref, ar2_ref, b2_ref), 0.01)
    pooled = jnp.mean(h, axis=0, keepdims=True)
    o = jnp.dot(pooled, wc_ref[...], preferred_element_type=jnp.float32)
    return _leaky(o + bc_ref[...], 0.01)


def _body(inst_ref, wimg_ref, bimg_ref, cw_ref, battn_ref,
          g_ref, ones_ref, g2_ref, w1_ref, al1_ref, ar1_ref, b1_ref,
          w2_ref, al2_ref, ar2_ref, b2_ref, wc_ref, bc_ref,
          out_ref, rep_ref, best_ref):
    step = pl.program_id(0)
    nsteps = pl.num_programs(0)

    @pl.when(step == 0)
    def _init():
        best_ref[...] = jnp.full(best_ref.shape, -jnp.inf, jnp.float32)

    cw = cw_ref[...]
    cent = cw[:_C, :]
    cn = jnp.sum(cent * cent, axis=1, keepdims=True)
    tile = inst_ref.shape[0]
    nchunks = 1
    chunk = tile // nchunks

    def _chunk(lo):
        xi = jnp.dot(inst_ref[pl.ds(lo, chunk), :], wimg_ref[...],
                     preferred_element_type=jnp.float32)
        xi = _leaky(xi + bimg_ref[...], 0.01)
        ts = _dot_t(cw, xi)
        t = ts[:_C, :]
        s = ts[_C:, :]
        xn = _dot_t(ones_ref[...], xi * xi)
        ns = xn - 2.0 * t + cn
        num = 1.0 / (1.0 + ns)
        ca = num / jnp.sum(num, axis=0, keepdims=True)
        logits = ca * s + battn_ref[...] + g_ref[0][:, lo:lo + chunk]
        col = jax.lax.broadcasted_iota(jnp.int32, logits.shape, 1)
        lmax = jnp.max(logits, axis=1, keepdims=True)
        lidx = jnp.min(jnp.where(logits == lmax, col, chunk),
                       axis=1, keepdims=True)
        sel = (col == lidx).astype(jnp.float32)
        return lmax, jnp.dot(sel, xi, preferred_element_type=jnp.float32)

    lmax, cand = _chunk(0)
    for k in range(1, nchunks):
        lm2, c2 = _chunk(k * chunk)
        upd = lm2 > lmax
        cand = jnp.where(upd, c2, cand)
        lmax = jnp.maximum(lmax, lm2)

    best = best_ref[...]
    improved = lmax > best
    best_ref[...] = jnp.maximum(best, lmax)
    rep_ref[...] = jnp.where(improved, cand, rep_ref[...])

    @pl.when(step == nsteps - 1)
    def _finish():
        out_ref[...] = _graph_stage(
            rep_ref[...], g2_ref[...], w1_ref, al1_ref, ar1_ref, b1_ref,
            w2_ref, al2_ref, ar2_ref, b2_ref, wc_ref, bc_ref)


def kernel(bags, W_img, b_img, centers, W_attn, b_attn,
           W1, al1, ar1, b1, W2, al2, ar2, b2, Wc, bc):
    inst = bags[0]
    n, d = inst.shape
    h = W1.shape[1]
    ncls = Wc.shape[1]

    tile = next((t for t in range(min(n, 2048), 7, -1)
                 if n % t == 0 and t % 8 == 0), None)
    pad = 0
    if tile is None:
        tile = 2048
        pad = (-n) % tile
    grid = (n + pad) // tile

    def _noise():
        gk = jax.random.key(1)
        g1 = jax.vmap(
            lambda i: jax.random.gumbel(jax.random.fold_in(gk, i), (n,),
                                        jnp.float32))(jnp.arange(_C))
        g1 = jnp.pad(g1, ((0, 0), (0, pad)), constant_values=-jnp.inf)
        g1 = g1.reshape(_C, grid, tile).transpose(1, 0, 2)
        g2 = jax.random.gumbel(jax.random.fold_in(gk, 1000), (_C, _C),
                               jnp.float32)
        return g1, g2

    try:
        with jax.ensure_compile_time_eval():
            g1, g2 = _noise()
    except Exception:
        g1, g2 = _noise()

    if pad:
        inst = jnp.pad(inst, ((0, pad), (0, 0)))

    fixed = lambda i: (0, 0)
    out = pl.pallas_call(
        _body,
        grid=(grid,),
        in_specs=[
            pl.BlockSpec((tile, d), lambda i: (i, 0)),
            pl.BlockSpec((d, d), fixed),
            pl.BlockSpec((1, d), fixed),
            pl.BlockSpec((_C + 1, d), fixed),
            pl.BlockSpec((1, 1), fixed),
            pl.BlockSpec((1, _C, tile), lambda i: (i, 0, 0)),
            pl.BlockSpec((1, d), fixed),
            pl.BlockSpec((_C, _C), fixed),
            pl.BlockSpec((d, h), fixed),
            pl.BlockSpec((1, h), fixed),
            pl.BlockSpec((1, h), fixed),
            pl.BlockSpec((1, h), fixed),
            pl.BlockSpec((h, h), fixed),
            pl.BlockSpec((1, h), fixed),
            pl.BlockSpec((1, h), fixed),
            pl.BlockSpec((1, h), fixed),
            pl.BlockSpec((h, ncls), fixed),
            pl.BlockSpec((1, ncls), fixed),
        ],
        out_specs=pl.BlockSpec((1, ncls), fixed),
        out_shape=jax.ShapeDtypeStruct((1, ncls), jnp.float32),
        scratch_shapes=[pltpu.VMEM((_C, d), jnp.float32),
                        pltpu.VMEM((_C, 1), jnp.float32)],
        compiler_params=pltpu.CompilerParams(
            dimension_semantics=("arbitrary",)),
    )(inst, W_img, b_img.reshape(1, d),
      jnp.concatenate([centers, W_attn.reshape(1, d)], axis=0),
      b_attn.reshape(1, 1), g1, jnp.ones((1, d), jnp.float32),
      g2, W1, al1.reshape(1, h), ar1.reshape(1, h), b1.reshape(1, h),
      W2, al2.reshape(1, h), ar2.reshape(1, h), b2.reshape(1, h),
      Wc, bc.reshape(1, ncls))
    return out[0]


# speedup vs baseline: 1.6541x; 1.1328x over previous
import jax
import jax.numpy as jnp
from jax.experimental import pallas as pl
from jax.experimental.pallas import tpu as pltpu

_C = 16


def _leaky(x, slope):
    return jnp.maximum(x, slope * x)


def _dot_t(a, b):
    return jax.lax.dot_general(a, b, (((1,), (1,)), ((), ())),
                               preferred_element_type=jnp.float32)


def _dot_c(a, b):
    return jax.lax.dot_general(a, b, (((0,), (0,)), ((), ())),
                               preferred_element_type=jnp.float32)


def _graph_stage(rep, g2, w1_ref, al1_ref, ar1_ref, b1_ref,
                 w2_ref, al2_ref, ar2_ref, b2_ref, wc_ref, bc_ref):
    rr = _dot_t(rep, rep)
    jio = jax.lax.broadcasted_iota(jnp.int32, (_C, _C), 1)
    iio = jax.lax.broadcasted_iota(jnp.int32, (_C, _C), 0)
    eye = (jio == iio).astype(jnp.float32)
    n2_row = jnp.sum(rr * eye, axis=0, keepdims=True)
    n2_col = jnp.sum(rr * eye, axis=1, keepdims=True)
    nrm_row = jnp.maximum(jnp.sqrt(n2_row), 1e-8)
    nrm_col = jnp.maximum(jnp.sqrt(n2_col), 1e-8)
    simg = rr / (nrm_col * nrm_row) + g2
    rmax = jnp.max(simg, axis=1, keepdims=True)
    dsel = jnp.min(jnp.where(simg == rmax, jio, _C),
                   axis=1, keepdims=True)
    mcnt = (jio == dsel).astype(jnp.float32) + eye
    mask = mcnt > 0.0

    def gat(h, w_ref, al_ref, ar_ref, b_ref):
        z = jnp.dot(h, w_ref[...], preferred_element_type=jnp.float32)
        el = jnp.sum(z * al_ref[...], axis=1, keepdims=True)
        er = _dot_t(ar_ref[...], z)
        e = _leaky(el + er, 0.2)
        m = jnp.max(jnp.where(mask, e, -jnp.inf), axis=0, keepdims=True)
        ex = jnp.where(mask, mcnt * jnp.exp(e - m), 0.0)
        a = ex / jnp.sum(ex, axis=0, keepdims=True)
        return _dot_c(a, z) + b_ref[...]

    h = _leaky(gat(rep, w1_ref, al1_ref, ar1_ref, b1_ref), 0.01)
    h = _leaky(gat(h, w2_ref, al2_ref, ar2_ref, b2_ref), 0.01)
    pooled = jnp.mean(h, axis=0, keepdims=True)
    o = jnp.dot(pooled, wc_ref[...], preferred_element_type=jnp.float32)
    return _leaky(o + bc_ref[...], 0.01)


def _body(inst_ref, wimg_ref, bimg_ref, cent_ref, wattn_ref, battn_ref,
          g_ref, g2_ref, w1_ref, al1_ref, ar1_ref, b1_ref,
          w2_ref, al2_ref, ar2_ref, b2_ref, wc_ref, bc_ref,
          out_ref, rep_ref, best_ref):
    step = pl.program_id(0)
    nsteps = pl.num_programs(0)

    @pl.when(step == 0)
    def _init():
        best_ref[...] = jnp.full(best_ref.shape, -jnp.inf, jnp.float32)

    cent = cent_ref[...]
    cw = jnp.concatenate([cent, wattn_ref[...]], axis=0)
    cn = jnp.sum(cent * cent, axis=1, keepdims=True)
    ones = jnp.full(wattn_ref.shape, 1.0, jnp.float32)
    tile = inst_ref.shape[0]
    nchunks = 1
    chunk = tile // nchunks

    def _chunk(lo):
        xi = jnp.dot(inst_ref[pl.ds(lo, chunk), :], wimg_ref[...],
                     preferred_element_type=jnp.float32)
        xi = _leaky(xi + bimg_ref[...], 0.01)
        ts = _dot_t(cw, xi)
        t = ts[:_C, :]
        s = ts[_C:, :]
        xn = _dot_t(ones, xi * xi)
        ns = xn - 2.0 * t + cn
        num = 1.0 / (1.0 + ns)
        ca = num / jnp.sum(num, axis=0, keepdims=True)
        logits = ca * s + battn_ref[...] + g_ref[0][:, lo:lo + chunk]
        col = jax.lax.broadcasted_iota(jnp.int32, logits.shape, 1)
        lmax = jnp.max(logits, axis=1, keepdims=True)
        lidx = jnp.min(jnp.where(logits == lmax, col, chunk),
                       axis=1, keepdims=True)
        sel = (col == lidx).astype(jnp.float32)
        return lmax, jnp.dot(sel, xi, preferred_element_type=jnp.float32)

    lmax, cand = _chunk(0)
    for k in range(1, nchunks):
        lm2, c2 = _chunk(k * chunk)
        upd = lm2 > lmax
        cand = jnp.where(upd, c2, cand)
        lmax = jnp.maximum(lmax, lm2)

    best = best_ref[...]
    improved = lmax > best
    best_ref[...] = jnp.maximum(best, lmax)
    rep_ref[...] = jnp.where(improved, cand, rep_ref[...])

    @pl.when(step == nsteps - 1)
    def _finish():
        out_ref[...] = _graph_stage(
            rep_ref[...], g2_ref[...], w1_ref, al1_ref, ar1_ref, b1_ref,
            w2_ref, al2_ref, ar2_ref, b2_ref, wc_ref, bc_ref)


def kernel(bags, W_img, b_img, centers, W_attn, b_attn,
           W1, al1, ar1, b1, W2, al2, ar2, b2, Wc, bc):
    inst = bags[0]
    n, d = inst.shape
    h = W1.shape[1]
    ncls = Wc.shape[1]

    tile = next((t for t in range(min(n, 2048), 7, -1)
                 if n % t == 0 and t % 8 == 0), None)
    pad = 0
    if tile is None:
        tile = 2048
        pad = (-n) % tile
    grid = (n + pad) // tile

    def _noise():
        gk = jax.random.key(1)
        g1 = jax.vmap(
            lambda i: jax.random.gumbel(jax.random.fold_in(gk, i), (n,),
                                        jnp.float32))(jnp.arange(_C))
        g1 = jnp.pad(g1, ((0, 0), (0, pad)), constant_values=-jnp.inf)
        g1 = g1.reshape(_C, grid, tile).transpose(1, 0, 2)
        g2 = jax.random.gumbel(jax.random.fold_in(gk, 1000), (_C, _C),
                               jnp.float32)
        return g1, g2

    try:
        with jax.ensure_compile_time_eval():
            g1, g2 = _noise()
    except Exception:
        g1, g2 = _noise()

    if pad:
        inst = jnp.pad(inst, ((0, pad), (0, 0)))

    fixed = lambda i: (0, 0)
    out = pl.pallas_call(
        _body,
        grid=(grid,),
        in_specs=[
            pl.BlockSpec((tile, d), lambda i: (i, 0)),
            pl.BlockSpec((d, d), fixed),
            pl.BlockSpec((1, d), fixed),
            pl.BlockSpec((_C, d), fixed),
            pl.BlockSpec((1, d), fixed),
            pl.BlockSpec((1, 1), fixed),
            pl.BlockSpec((1, _C, tile), lambda i: (i, 0, 0)),
            pl.BlockSpec((_C, _C), fixed),
            pl.BlockSpec((d, h), fixed),
            pl.BlockSpec((1, h), fixed),
            pl.BlockSpec((1, h), fixed),
            pl.BlockSpec((1, h), fixed),
            pl.BlockSpec((h, h), fixed),
            pl.BlockSpec((1, h), fixed),
            pl.BlockSpec((1, h), fixed),
            pl.BlockSpec((1, h), fixed),
            pl.BlockSpec((h, ncls), fixed),
            pl.BlockSpec((1, ncls), fixed),
        ],
        out_specs=pl.BlockSpec((1, ncls), fixed),
        out_shape=jax.ShapeDtypeStruct((1, ncls), jnp.float32),
        scratch_shapes=[pltpu.VMEM((_C, d), jnp.float32),
                        pltpu.VMEM((_C, 1), jnp.float32)],
        compiler_params=pltpu.CompilerParams(
            dimension_semantics=("arbitrary",)),
    )(inst, W_img, b_img.reshape(1, d), centers, W_attn.reshape(1, d),
      b_attn.reshape(1, 1), g1,
      g2, W1, al1.reshape(1, h), ar1.reshape(1, h), b1.reshape(1, h),
      W2, al2.reshape(1, h), ar2.reshape(1, h), b2.reshape(1, h),
      Wc, bc.reshape(1, ncls))
    return out[0]
